# baseline (speedup 1.0000x reference)
import functools
import math

import jax
import jax.numpy as jnp
from jax import lax
from jax.experimental import pallas as pl
from jax.experimental.pallas import tpu as pltpu
from jax.experimental.pallas import tpu_sc as plsc

NC = 2
NS = 16
NW = NC * NS
BLK = 128

f32 = jnp.float32
i32 = jnp.int32


def _mesh():
    return plsc.VectorSubcoreMesh(
        core_axis_name="c", subcore_axis_name="s", num_cores=NC, num_subcores=NS
    )


_SC_PARAMS = pltpu.CompilerParams(use_tc_tiling_on_sc=False)


def _make_hist(n_pad, nbf, nb0):
    rpt = n_pad // NS
    nb1 = nbf - nb0

    @functools.partial(
        pl.kernel,
        out_type=[
            jax.ShapeDtypeStruct((NC, n_pad, 16), f32),
            jax.ShapeDtypeStruct((NC, n_pad, 16), f32),
        ],
        mesh=_mesh(),
        scratch_types=[
            pltpu.VMEM((max(nb0, nbf - nb0), BLK), i32),
            pltpu.VMEM((max(nb0, nbf - nb0), BLK), i32),
            pltpu.VMEM((BLK, 16), f32),
            pltpu.VMEM_SHARED((n_pad, 16), f32),
            pltpu.VMEM_SHARED((n_pad, 16), f32),
            pltpu.SemaphoreType.DMA,
            pltpu.SemaphoreType.DMA,
        ],
        compiler_params=_SC_PARAMS,
    )
    def hist(src_hbm, dst_hbm, ones_hbm, zeros_hbm, hs_hbm, hd_hbm,
             src_v, dst_v, ones_v, acc_s, acc_d, sem_s, sem_d):
        cid = lax.axis_index("c")
        sid = lax.axis_index("s")
        base = sid * rpt
        c0 = cid == 0

        def _idx_copies(lo, cnt):
            return (
                pltpu.make_async_copy(src_hbm.at[sid, pl.ds(lo, cnt)],
                                      src_v.at[pl.ds(0, cnt)], sem_s),
                pltpu.make_async_copy(dst_hbm.at[sid, pl.ds(lo, cnt)],
                                      dst_v.at[pl.ds(0, cnt)], sem_d),
            )

        @pl.when(c0)
        def _():
            for cp in _idx_copies(0, nb0):
                cp.start()

        @pl.when(jnp.logical_not(c0))
        def _():
            for cp in _idx_copies(nb0, nb1):
                cp.start()

        pltpu.sync_copy(ones_hbm, ones_v)
        pltpu.sync_copy(zeros_hbm, acc_s.at[pl.ds(base, rpt)])
        pltpu.sync_copy(zeros_hbm, acc_d.at[pl.ds(base, rpt)])

        @pl.when(c0)
        def _():
            for cp in _idx_copies(0, nb0):
                cp.wait()

        @pl.when(jnp.logical_not(c0))
        def _():
            for cp in _idx_copies(nb0, nb1):
                cp.wait()

        plsc.subcore_barrier()
        count = jnp.where(c0, nb0, nb1)

        @pl.loop(0, count)
        def _(blk):
            pltpu.async_copy(ones_v, acc_s.at[src_v.at[blk]], sem_s, add=True)
            pltpu.async_copy(ones_v, acc_d.at[dst_v.at[blk]], sem_d, add=True)
            pltpu.make_async_copy(ones_v, acc_s.at[src_v.at[blk]], sem_s).wait()
            pltpu.make_async_copy(ones_v, acc_d.at[dst_v.at[blk]], sem_d).wait()

        plsc.subcore_barrier()
        pltpu.sync_copy(acc_s.at[pl.ds(base, rpt)], hs_hbm.at[cid, pl.ds(base, rpt)])
        pltpu.sync_copy(acc_d.at[pl.ds(base, rpt)], hd_hbm.at[cid, pl.ds(base, rpt)])

    return hist


NBUF = 4


def _ring_loop(start, count, tbl_hbm, src_v, dst_v, rows_v, acc, sems, nbuf):
    for j in range(nbuf):
        pltpu.async_copy(tbl_hbm.at[src_v.at[start + j]], rows_v.at[j], sems[j])

    @pl.loop(0, count, step=nbuf)
    def _(b):
        for j in range(nbuf):
            blk = start + b + j
            pltpu.make_async_copy(
                tbl_hbm.at[src_v.at[blk]], rows_v.at[j], sems[j]
            ).wait()
            pltpu.sync_copy(rows_v.at[j], acc.at[dst_v.at[blk]], add=True)

            @pl.when(b + j + nbuf < count)
            def _():
                pltpu.async_copy(tbl_hbm.at[src_v.at[blk + nbuf]], rows_v.at[j],
                                 sems[j])


RING = 5
LOOK = 3


def _make_agg_feat(n_pad, d, nb):
    rpt = n_pad // NS

    @functools.partial(
        pl.kernel,
        out_type=jax.ShapeDtypeStruct((NC, n_pad, d), f32),
        mesh=_mesh(),
        scratch_types=[
            pltpu.VMEM((nb, BLK), i32),
            pltpu.VMEM((nb, BLK), i32),
            pltpu.VMEM((RING, BLK, d), f32),
            pltpu.VMEM_SHARED((n_pad, d), f32),
        ] + [pltpu.SemaphoreType.DMA] * (2 * RING),
        compiler_params=_SC_PARAMS,
    )
    def agg(src_hbm, dst_hbm, tbl_hbm, zeros_hbm, out_hbm,
            src_v, dst_v, rows_v, acc, *sems):
        gsems, ssems = sems[:RING], sems[RING:]
        cid = lax.axis_index("c")
        sid = lax.axis_index("s")
        base = sid * rpt
        cp_s = pltpu.async_copy(src_hbm.at[cid, sid], src_v, gsems[0])
        cp_d = pltpu.async_copy(dst_hbm.at[sid], dst_v, gsems[1])
        pltpu.sync_copy(zeros_hbm, acc.at[pl.ds(base, rpt)])
        cp_s.wait()
        cp_d.wait()
        plsc.subcore_barrier()

        for j in range(LOOK):
            pltpu.async_copy(tbl_hbm.at[src_v.at[j]], rows_v.at[j], gsems[j])

        @pl.loop(0, nb, step=RING)
        def _(b):
            for j in range(RING):
                blk = b + j
                jj = (j + LOOK) % RING
                pltpu.make_async_copy(
                    tbl_hbm.at[src_v.at[blk]], rows_v.at[j], gsems[j]
                ).wait()
                pltpu.async_copy(rows_v.at[j], acc.at[dst_v.at[blk]],
                                 ssems[j], add=True)
                ahead = blk + LOOK

                @pl.when(ahead < nb)
                def _():
                    @pl.when(ahead >= RING)
                    def _():
                        pltpu.make_async_copy(
                            rows_v.at[jj], acc.at[dst_v.at[ahead - RING]],
                            ssems[jj],
                        ).wait()

                    pltpu.async_copy(tbl_hbm.at[src_v.at[ahead]],
                                     rows_v.at[jj], gsems[jj])

        for j in range(RING):
            pltpu.make_async_copy(
                rows_v.at[j], acc.at[dst_v.at[nb - RING + j]], ssems[j]
            ).wait()

        plsc.subcore_barrier()
        pltpu.sync_copy(acc.at[pl.ds(base, rpt)], out_hbm.at[cid, pl.ds(base, rpt)])

    return agg


def _make_agg_edge(n_pad, d, nbf, nb0, nbuf=8):
    rpt = n_pad // NS
    nb1 = nbf - nb0
    nmax = max(nb0, nb1)

    @functools.partial(
        pl.kernel,
        out_type=jax.ShapeDtypeStruct((NC, n_pad, d), f32),
        mesh=_mesh(),
        scratch_types=[
            pltpu.VMEM((nmax, BLK), i32),
            pltpu.VMEM((nmax, BLK), i32),
            pltpu.VMEM((nbuf, BLK, d), f32),
            pltpu.VMEM_SHARED((n_pad, d), f32),
        ] + [pltpu.SemaphoreType.DMA] * nbuf,
        compiler_params=_SC_PARAMS,
    )
    def agg(src_hbm, dst_hbm, tbl_hbm, zeros_hbm, out_hbm,
            src_v, dst_v, rows_v, acc, *sems):
        cid = lax.axis_index("c")
        sid = lax.axis_index("s")
        base = sid * rpt
        c0 = cid == 0

        def _idx_copies(lo, cnt):
            return (
                pltpu.make_async_copy(src_hbm.at[sid, pl.ds(lo, cnt)],
                                      src_v.at[pl.ds(0, cnt)], sems[0]),
                pltpu.make_async_copy(dst_hbm.at[sid, pl.ds(lo, cnt)],
                                      dst_v.at[pl.ds(0, cnt)], sems[1]),
            )

        @pl.when(c0)
        def _():
            for cp in _idx_copies(0, nb0):
                cp.start()

        @pl.when(jnp.logical_not(c0))
        def _():
            for cp in _idx_copies(nb0, nb1):
                cp.start()

        pltpu.sync_copy(zeros_hbm, acc.at[pl.ds(base, rpt)])

        @pl.when(c0)
        def _():
            for cp in _idx_copies(0, nb0):
                cp.wait()

        @pl.when(jnp.logical_not(c0))
        def _():
            for cp in _idx_copies(nb0, nb1):
                cp.wait()

        plsc.subcore_barrier()
        count = jnp.where(c0, nb0, nb1)
        _ring_loop(0, count, tbl_hbm, src_v, dst_v, rows_v, acc, sems, nbuf)
        plsc.subcore_barrier()
        pltpu.sync_copy(acc.at[pl.ds(base, rpt)], out_hbm.at[cid, pl.ds(base, rpt)])

    return agg


def _make_tc_feat(dh):
    def _tc_feat(x_ref, w_ref, hs_ref, o_ref):
        hs = hs_ref[...]
        deg = 1.0 + hs[0, :, 0] + hs[1, :, 0]
        xw = jnp.dot(x_ref[...], w_ref[...], preferred_element_type=f32,
                     precision=lax.Precision.HIGHEST)
        h1 = xw * lax.rsqrt(deg)[:, None]
        o_ref[...] = jnp.stack([h1[:, :dh], h1[:, dh:]])

    return _tc_feat


def _make_tc_mid():
    def _tc_mid(a_ref, h1_ref, hs_ref, hd_ref, b1_ref, w2_ref, o_ref):
        a = a_ref[...]
        h1s = h1_ref[...]
        h1 = jnp.concatenate([h1s[0], h1s[1]], axis=1)
        agg = jnp.concatenate([a[0], a[1]], axis=1) + h1
        hd = hd_ref[...]
        deg_in = 1.0 + hd[0, :, 0] + hd[1, :, 0]
        y = jnp.maximum(agg * lax.rsqrt(deg_in)[:, None] + b1_ref[...], 0.0)
        s = jnp.sum(y * w2_ref[...], axis=1)
        hs = hs_ref[...]
        deg_out = 1.0 + hs[0, :, 0] + hs[1, :, 0]
        h2 = s * lax.rsqrt(deg_out)
        col = lax.broadcasted_iota(i32, o_ref.shape, 1)
        o_ref[...] = jnp.where(col == 0, h2[:, None], 0.0)

    return _tc_mid


def _make_tc_out(n):
    def _tc_out(a2_ref, h2p_ref, hd_ref, b2_ref, o_ref):
        a2 = a2_ref[...]
        s = a2[0, :, 0] + a2[1, :, 0] + h2p_ref[...][:, 0]
        hd = hd_ref[...]
        deg_in = 1.0 + hd[0, :, 0] + hd[1, :, 0]
        o_ref[...] = (s * lax.rsqrt(deg_in))[:n, None] + b2_ref[...]

    return _tc_out


def kernel(in_feat, edge_index, W1, b1, W2, b2):
    n, d_in = in_feat.shape
    d_h = W1.shape[1]
    e = edge_index.shape[1]

    n_pad = pl.cdiv(n, BLK) * BLK
    unit = math.lcm(NW * NBUF * BLK, NS * RING * BLK)
    e_pad = pl.cdiv(e, unit) * unit
    nb = e_pad // (NW * BLK)
    trash = n_pad - 1

    nbf = e_pad // (NS * BLK)
    dh = d_h // NC

    src = edge_index[0].astype(i32)
    dst = edge_index[1].astype(i32)
    fill = jnp.full((e_pad - e,), trash, i32)
    src_f = jnp.concatenate([src, fill])
    dst_f = jnp.concatenate([dst, fill])
    src16 = src_f.reshape(NS, nbf, BLK)
    src_feat = jnp.stack([src16, src16 + n_pad])
    dst16 = dst_f.reshape(NS, nbf, BLK)

    x_pad = jnp.pad(in_feat, ((0, n_pad - n), (0, 0)))
    ones16 = jnp.ones((BLK, 16), f32)
    zeros16 = jnp.zeros((n_pad // NS, 16), f32)
    zeros_dh = jnp.zeros((n_pad // NS, dh), f32)

    nb0h = ((int(nbf * 0.55) + 1) // 2) * 2
    hs, hd = _make_hist(n_pad, nbf, nb0h)(src16, dst16, ones16, zeros16)

    RB = n_pad // 8
    nrb = n_pad // RB
    h1s = pl.pallas_call(
        _make_tc_feat(dh),
        grid=(nrb,),
        in_specs=[
            pl.BlockSpec((RB, d_in), lambda i: (i, 0)),
            pl.BlockSpec((d_in, d_h), lambda i: (0, 0)),
            pl.BlockSpec((NC, RB, 16), lambda i: (0, i, 0)),
        ],
        out_specs=pl.BlockSpec((NC, RB, dh), lambda i: (0, i, 0)),
        out_shape=jax.ShapeDtypeStruct((NC, n_pad, dh), f32),
    )(x_pad, W1, hs)

    agg1 = _make_agg_feat(n_pad, dh, nbf)(
        src_feat, dst16, h1s.reshape(NC * n_pad, dh), zeros_dh)

    h2p = pl.pallas_call(
        _make_tc_mid(),
        grid=(nrb,),
        in_specs=[
            pl.BlockSpec((NC, RB, dh), lambda i: (0, i, 0)),
            pl.BlockSpec((NC, RB, dh), lambda i: (0, i, 0)),
            pl.BlockSpec((NC, RB, 16), lambda i: (0, i, 0)),
            pl.BlockSpec((NC, RB, 16), lambda i: (0, i, 0)),
            pl.BlockSpec((1, d_h), lambda i: (0, 0)),
            pl.BlockSpec((1, d_h), lambda i: (0, 0)),
        ],
        out_specs=pl.BlockSpec((RB, 16), lambda i: (i, 0)),
        out_shape=jax.ShapeDtypeStruct((n_pad, 16), f32),
    )(agg1, h1s, hs, hd, b1.reshape(1, d_h), W2.reshape(1, d_h))

    nb0 = (int(nbf * 0.70) // NBUF) * NBUF
    agg2 = _make_agg_edge(n_pad, 16, nbf, nb0)(src16, dst16, h2p, zeros16)

    out = pl.pallas_call(
        _make_tc_out(n), out_shape=jax.ShapeDtypeStruct((n, 1), f32),
    )(agg2, h2p, hd, b2.reshape(1, 1))
    return out


# speedup vs baseline: 1.0230x; 1.0230x over previous
import functools
import math

import jax
import jax.numpy as jnp
from jax import lax
from jax.experimental import pallas as pl
from jax.experimental.pallas import tpu as pltpu
from jax.experimental.pallas import tpu_sc as plsc

NC = 2
NS = 16
NW = NC * NS
BLK = 128

f32 = jnp.float32
i32 = jnp.int32


def _mesh():
    return plsc.VectorSubcoreMesh(
        core_axis_name="c", subcore_axis_name="s", num_cores=NC, num_subcores=NS
    )


_SC_PARAMS = pltpu.CompilerParams(use_tc_tiling_on_sc=False)


def _make_hist(n_pad, nbf, nb0):
    rpt = n_pad // NS
    nb1 = nbf - nb0

    @functools.partial(
        pl.kernel,
        out_type=[
            jax.ShapeDtypeStruct((NC, n_pad, 16), f32),
            jax.ShapeDtypeStruct((NC, n_pad, 16), f32),
        ],
        mesh=_mesh(),
        scratch_types=[
            pltpu.VMEM((max(nb0, nbf - nb0), BLK), i32),
            pltpu.VMEM((max(nb0, nbf - nb0), BLK), i32),
            pltpu.VMEM((BLK, 16), f32),
            pltpu.VMEM_SHARED((n_pad, 16), f32),
            pltpu.VMEM_SHARED((n_pad, 16), f32),
            pltpu.SemaphoreType.DMA,
            pltpu.SemaphoreType.DMA,
        ],
        compiler_params=_SC_PARAMS,
    )
    def hist(src_hbm, dst_hbm, ones_hbm, zeros_hbm, hs_hbm, hd_hbm,
             src_v, dst_v, ones_v, acc_s, acc_d, sem_s, sem_d):
        cid = lax.axis_index("c")
        sid = lax.axis_index("s")
        base = sid * rpt
        c0 = cid == 0

        def _idx_copies(lo, cnt):
            return (
                pltpu.make_async_copy(src_hbm.at[sid, pl.ds(lo, cnt)],
                                      src_v.at[pl.ds(0, cnt)], sem_s),
                pltpu.make_async_copy(dst_hbm.at[sid, pl.ds(lo, cnt)],
                                      dst_v.at[pl.ds(0, cnt)], sem_d),
            )

        @pl.when(c0)
        def _():
            for cp in _idx_copies(0, nb0):
                cp.start()

        @pl.when(jnp.logical_not(c0))
        def _():
            for cp in _idx_copies(nb0, nb1):
                cp.start()

        pltpu.sync_copy(ones_hbm, ones_v)
        pltpu.sync_copy(zeros_hbm, acc_s.at[pl.ds(base, rpt)])
        pltpu.sync_copy(zeros_hbm, acc_d.at[pl.ds(base, rpt)])

        @pl.when(c0)
        def _():
            for cp in _idx_copies(0, nb0):
                cp.wait()

        @pl.when(jnp.logical_not(c0))
        def _():
            for cp in _idx_copies(nb0, nb1):
                cp.wait()

        plsc.subcore_barrier()
        count = jnp.where(c0, nb0, nb1)

        @pl.loop(0, count)
        def _(blk):
            pltpu.async_copy(ones_v, acc_s.at[src_v.at[blk]], sem_s, add=True)
            pltpu.async_copy(ones_v, acc_d.at[dst_v.at[blk]], sem_d, add=True)
            pltpu.make_async_copy(ones_v, acc_s.at[src_v.at[blk]], sem_s).wait()
            pltpu.make_async_copy(ones_v, acc_d.at[dst_v.at[blk]], sem_d).wait()

        plsc.subcore_barrier()
        pltpu.sync_copy(acc_s.at[pl.ds(base, rpt)], hs_hbm.at[cid, pl.ds(base, rpt)])
        pltpu.sync_copy(acc_d.at[pl.ds(base, rpt)], hd_hbm.at[cid, pl.ds(base, rpt)])

    return hist


NBUF = 4


def _ring_loop(start, count, tbl_hbm, src_v, dst_v, rows_v, acc, sems, nbuf):
    for j in range(nbuf):
        pltpu.async_copy(tbl_hbm.at[src_v.at[start + j]], rows_v.at[j], sems[j])

    @pl.loop(0, count, step=nbuf)
    def _(b):
        for j in range(nbuf):
            blk = start + b + j
            pltpu.make_async_copy(
                tbl_hbm.at[src_v.at[blk]], rows_v.at[j], sems[j]
            ).wait()
            pltpu.sync_copy(rows_v.at[j], acc.at[dst_v.at[blk]], add=True)

            @pl.when(b + j + nbuf < count)
            def _():
                pltpu.async_copy(tbl_hbm.at[src_v.at[blk + nbuf]], rows_v.at[j],
                                 sems[j])


RING = 5
LOOK = 3


def _make_agg_feat(n_pad, d, nb):
    rpt = n_pad // NS

    @functools.partial(
        pl.kernel,
        out_type=jax.ShapeDtypeStruct((NC, n_pad, d), f32),
        mesh=_mesh(),
        scratch_types=[
            pltpu.VMEM((nb, BLK), i32),
            pltpu.VMEM((nb, BLK), i32),
            pltpu.VMEM((RING, BLK, d), f32),
            pltpu.VMEM_SHARED((n_pad, d), f32),
        ] + [pltpu.SemaphoreType.DMA] * (2 * RING),
        compiler_params=_SC_PARAMS,
    )
    def agg(src_hbm, dst_hbm, tbl_hbm, zeros_hbm, out_hbm,
            src_v, dst_v, rows_v, acc, *sems):
        gsems, ssems = sems[:RING], sems[RING:]
        cid = lax.axis_index("c")
        sid = lax.axis_index("s")
        base = sid * rpt
        cp_s = pltpu.async_copy(src_hbm.at[cid, sid], src_v, gsems[0])
        cp_d = pltpu.async_copy(dst_hbm.at[sid], dst_v, gsems[1])
        pltpu.sync_copy(zeros_hbm, acc.at[pl.ds(base, rpt)])
        cp_s.wait()
        cp_d.wait()
        plsc.subcore_barrier()

        for j in range(LOOK):
            pltpu.async_copy(tbl_hbm.at[src_v.at[j]], rows_v.at[j], gsems[j])

        @pl.loop(0, nb, step=RING)
        def _(b):
            for j in range(RING):
                blk = b + j
                jj = (j + LOOK) % RING
                pltpu.make_async_copy(
                    tbl_hbm.at[src_v.at[blk]], rows_v.at[j], gsems[j]
                ).wait()
                pltpu.async_copy(rows_v.at[j], acc.at[dst_v.at[blk]],
                                 ssems[j], add=True)
                ahead = blk + LOOK

                @pl.when(ahead < nb)
                def _():
                    @pl.when(ahead >= RING)
                    def _():
                        pltpu.make_async_copy(
                            rows_v.at[jj], acc.at[dst_v.at[ahead - RING]],
                            ssems[jj],
                        ).wait()

                    pltpu.async_copy(tbl_hbm.at[src_v.at[ahead]],
                                     rows_v.at[jj], gsems[jj])

        for j in range(RING):
            pltpu.make_async_copy(
                rows_v.at[j], acc.at[dst_v.at[nb - RING + j]], ssems[j]
            ).wait()

        plsc.subcore_barrier()
        pltpu.sync_copy(acc.at[pl.ds(base, rpt)], out_hbm.at[cid, pl.ds(base, rpt)])

    return agg


def _make_agg_edge(n_pad, d, nbf, nb0, nbuf=8):
    rpt = n_pad // NS
    nb1 = nbf - nb0
    nmax = max(nb0, nb1)

    @functools.partial(
        pl.kernel,
        out_type=jax.ShapeDtypeStruct((NC, n_pad, d), f32),
        mesh=_mesh(),
        scratch_types=[
            pltpu.VMEM((nmax, BLK), i32),
            pltpu.VMEM((nmax, BLK), i32),
            pltpu.VMEM((nbuf, BLK, d), f32),
            pltpu.VMEM_SHARED((n_pad, d), f32),
        ] + [pltpu.SemaphoreType.DMA] * nbuf,
        compiler_params=_SC_PARAMS,
    )
    def agg(src_hbm, dst_hbm, tbl_hbm, zeros_hbm, out_hbm,
            src_v, dst_v, rows_v, acc, *sems):
        cid = lax.axis_index("c")
        sid = lax.axis_index("s")
        base = sid * rpt
        c0 = cid == 0

        def _idx_copies(lo, cnt):
            return (
                pltpu.make_async_copy(src_hbm.at[sid, pl.ds(lo, cnt)],
                                      src_v.at[pl.ds(0, cnt)], sems[0]),
                pltpu.make_async_copy(dst_hbm.at[sid, pl.ds(lo, cnt)],
                                      dst_v.at[pl.ds(0, cnt)], sems[1]),
            )

        @pl.when(c0)
        def _():
            for cp in _idx_copies(0, nb0):
                cp.start()

        @pl.when(jnp.logical_not(c0))
        def _():
            for cp in _idx_copies(nb0, nb1):
                cp.start()

        pltpu.sync_copy(zeros_hbm, acc.at[pl.ds(base, rpt)])

        @pl.when(c0)
        def _():
            for cp in _idx_copies(0, nb0):
                cp.wait()

        @pl.when(jnp.logical_not(c0))
        def _():
            for cp in _idx_copies(nb0, nb1):
                cp.wait()

        plsc.subcore_barrier()
        count = jnp.where(c0, nb0, nb1)
        _ring_loop(0, count, tbl_hbm, src_v, dst_v, rows_v, acc, sems, nbuf)
        plsc.subcore_barrier()
        pltpu.sync_copy(acc.at[pl.ds(base, rpt)], out_hbm.at[cid, pl.ds(base, rpt)])

    return agg


def _tc_feat(x_ref, w_ref, hs_ref, o_ref):
    hs = hs_ref[...]
    deg = 1.0 + hs[0, :, 0] + hs[1, :, 0]
    xw = jnp.dot(x_ref[...], w_ref[...][0], preferred_element_type=f32,
                 precision=lax.Precision.HIGHEST)
    o_ref[...] = xw * lax.rsqrt(deg)[:, None]


def _make_tc_mid():
    def _tc_mid(a_ref, h1a_ref, h1b_ref, hs_ref, hd_ref, b1_ref, w2_ref, o_ref):
        a = a_ref[...]
        h1 = jnp.concatenate([h1a_ref[...], h1b_ref[...]], axis=1)
        agg = jnp.concatenate([a[0], a[1]], axis=1) + h1
        hd = hd_ref[...]
        deg_in = 1.0 + hd[0, :, 0] + hd[1, :, 0]
        y = jnp.maximum(agg * lax.rsqrt(deg_in)[:, None] + b1_ref[...], 0.0)
        s = jnp.sum(y * w2_ref[...], axis=1)
        hs = hs_ref[...]
        deg_out = 1.0 + hs[0, :, 0] + hs[1, :, 0]
        h2 = s * lax.rsqrt(deg_out)
        col = lax.broadcasted_iota(i32, o_ref.shape, 1)
        o_ref[...] = jnp.where(col == 0, h2[:, None], 0.0)

    return _tc_mid


def _make_tc_out(n):
    def _tc_out(a2_ref, h2p_ref, hd_ref, b2_ref, o_ref):
        a2 = a2_ref[...]
        s = a2[0, :, 0] + a2[1, :, 0] + h2p_ref[...][:, 0]
        hd = hd_ref[...]
        deg_in = 1.0 + hd[0, :, 0] + hd[1, :, 0]
        o_ref[...] = (s * lax.rsqrt(deg_in))[:n, None] + b2_ref[...]

    return _tc_out


def kernel(in_feat, edge_index, W1, b1, W2, b2):
    n, d_in = in_feat.shape
    d_h = W1.shape[1]
    e = edge_index.shape[1]

    n_pad = pl.cdiv(n, BLK) * BLK
    unit = math.lcm(NW * NBUF * BLK, NS * RING * BLK)
    e_pad = pl.cdiv(e, unit) * unit
    nb = e_pad // (NW * BLK)
    trash = n_pad - 1

    nbf = e_pad // (NS * BLK)
    dh = d_h // NC

    src = edge_index[0].astype(i32)
    dst = edge_index[1].astype(i32)
    fill = jnp.full((e_pad - e,), trash, i32)
    src_f = jnp.concatenate([src, fill])
    dst_f = jnp.concatenate([dst, fill])
    src16 = src_f.reshape(NS, nbf, BLK)
    src_feat = jnp.stack([src16, src16 + n_pad])
    dst16 = dst_f.reshape(NS, nbf, BLK)

    x_pad = jnp.pad(in_feat, ((0, n_pad - n), (0, 0)))
    ones16 = jnp.ones((BLK, 16), f32)
    zeros16 = jnp.zeros((n_pad // NS, 16), f32)
    zeros_dh = jnp.zeros((n_pad // NS, dh), f32)

    nb0h = ((int(nbf * 0.55) + 1) // 2) * 2
    hs, hd = _make_hist(n_pad, nbf, nb0h)(src16, dst16, ones16, zeros16)

    RB = n_pad // 8
    nrb = n_pad // RB
    h1s = pl.pallas_call(
        _tc_feat,
        grid=(NC, nrb),
        in_specs=[
            pl.BlockSpec((RB, d_in), lambda c, i: (i, 0)),
            pl.BlockSpec((1, d_in, dh), lambda c, i: (c, 0, 0)),
            pl.BlockSpec((NC, RB, 16), lambda c, i: (0, i, 0)),
        ],
        out_specs=pl.BlockSpec((RB, dh), lambda c, i: (c * nrb + i, 0)),
        out_shape=jax.ShapeDtypeStruct((NC * n_pad, dh), f32),
    )(x_pad, jnp.stack([W1[:, :dh], W1[:, dh:]]), hs)

    agg1 = _make_agg_feat(n_pad, dh, nbf)(src_feat, dst16, h1s, zeros_dh)

    h2p = pl.pallas_call(
        _make_tc_mid(),
        grid=(nrb,),
        in_specs=[
            pl.BlockSpec((NC, RB, dh), lambda i: (0, i, 0)),
            pl.BlockSpec((RB, dh), lambda i: (i, 0)),
            pl.BlockSpec((RB, dh), lambda i: (nrb + i, 0)),
            pl.BlockSpec((NC, RB, 16), lambda i: (0, i, 0)),
            pl.BlockSpec((NC, RB, 16), lambda i: (0, i, 0)),
            pl.BlockSpec((1, d_h), lambda i: (0, 0)),
            pl.BlockSpec((1, d_h), lambda i: (0, 0)),
        ],
        out_specs=pl.BlockSpec((RB, 16), lambda i: (i, 0)),
        out_shape=jax.ShapeDtypeStruct((n_pad, 16), f32),
    )(agg1, h1s, h1s, hs, hd, b1.reshape(1, d_h), W2.reshape(1, d_h))

    nb0 = (int(nbf * 0.70) // NBUF) * NBUF
    agg2 = _make_agg_edge(n_pad, 16, nbf, nb0)(src16, dst16, h2p, zeros16)

    out = pl.pallas_call(
        _make_tc_out(n), out_shape=jax.ShapeDtypeStruct((n, 1), f32),
    )(agg2, h2p, hd, b2.reshape(1, 1))
    return out
